# Initial kernel scaffold; baseline (speedup 1.0000x reference)
import jax
import jax.numpy as jnp
from jax.experimental import pallas as pl


def kernel(n_mu, n_sigma2, e_mu, e_sigma2, batch_node_key, batch_node_value, batch_edge_key, batch_edge_value):
    raise NotImplementedError("write your pallas kernel here")


# speedup vs baseline: 64.9639x; 64.9639x over previous
import functools

import jax
import jax.numpy as jnp
from jax import lax
from jax.experimental import pallas as pl
from jax.experimental.pallas import tpu as pltpu
from jax.experimental.pallas import tpu_sc as plsc

EPS = 1.0
LAMB = 0.5

B = 2_000_000
LANES = 128
R = B // LANES
CHR = 25
CHE = CHR * LANES
NCH = R // CHR
NC, NS = 2, 16
NW = NC * NS
TPW = (NCH + NW - 1) // NW


def _sc_gather_kernel(n_mu, n_s2, e_mu, e_s2, bnk, bek,
                      g_nmu, g_ns2, g_emu, g_es2,
                      keys_v, ga_v, gb_v, sem):
    wid = lax.axis_index("s") * NC + lax.axis_index("c")

    for keys_hbm, tab_a, tab_b, out_a, out_b in (
        (bnk, n_mu, n_s2, g_nmu, g_ns2),
        (bek, e_mu, e_s2, g_emu, g_es2),
    ):
        def body(t, _, keys_hbm=keys_hbm, tab_a=tab_a, tab_b=tab_b,
                 out_a=out_a, out_b=out_b):
            ch = t * NW + wid

            @pl.when(ch < NCH)
            def _():
                base = ch * CHE
                pltpu.sync_copy(keys_hbm.at[pl.ds(base, CHE)], keys_v)
                copies = []
                for j in range(CHR):
                    krow = keys_v.at[pl.ds(j * LANES, LANES)]
                    copies.append(pltpu.async_copy(
                        tab_a.at[krow], ga_v.at[pl.ds(j * LANES, LANES)], sem))
                    copies.append(pltpu.async_copy(
                        tab_b.at[krow], gb_v.at[pl.ds(j * LANES, LANES)], sem))
                for c in copies:
                    c.wait()
                pltpu.sync_copy(ga_v, out_a.at[pl.ds(base, CHE)])
                pltpu.sync_copy(gb_v, out_b.at[pl.ds(base, CHE)])

            return 0

        lax.fori_loop(0, TPW, body, 0)


def _sc_gather(n_mu, n_s2, e_mu, e_s2, bnk, bek):
    f32 = jnp.float32
    out = jax.ShapeDtypeStruct((B,), f32)
    return pl.kernel(
        _sc_gather_kernel,
        out_type=(out, out, out, out),
        mesh=plsc.VectorSubcoreMesh(core_axis_name="c", subcore_axis_name="s",
                                    num_cores=NC, num_subcores=NS),
        scratch_types=(
            pltpu.VMEM((CHE,), jnp.int32),
            pltpu.VMEM((CHE,), f32),
            pltpu.VMEM((CHE,), f32),
            pltpu.SemaphoreType.DMA,
        ),
    )(n_mu, n_s2, e_mu, e_s2, bnk, bek)


TC_BLK = 1024
TC_GRID = (R + TC_BLK - 1) // TC_BLK


def _tc_loss_kernel(nmu, ns2, vn, emu, es2, ve, out_ref):
    i = pl.program_id(0)
    rows = lax.broadcasted_iota(jnp.int32, (TC_BLK, LANES), 0) + i * TC_BLK
    mask = rows < R

    tn = 0.5 * jnp.log(EPS + ns2[...]) + (vn[...] - nmu[...]) ** 2 / (EPS + ns2[...])
    te = 0.5 * jnp.log(EPS + es2[...]) + (ve[...] - emu[...]) ** 2 / (EPS + es2[...])
    part = jnp.sum(jnp.where(mask, LAMB * tn + (1.0 - LAMB) * te, 0.0))

    @pl.when(i == 0)
    def _():
        out_ref[0, 0] = 0.0

    out_ref[0, 0] += part


def _tc_loss(g_nmu, g_ns2, vn_r, g_emu, g_es2, ve_r):
    spec = pl.BlockSpec((TC_BLK, LANES), lambda i: (i, 0))
    out = pl.pallas_call(
        _tc_loss_kernel,
        grid=(TC_GRID,),
        in_specs=[spec] * 6,
        out_specs=pl.BlockSpec(memory_space=pltpu.SMEM),
        out_shape=jax.ShapeDtypeStruct((1, 1), jnp.float32),
    )(g_nmu, g_ns2, vn_r, g_emu, g_es2, ve_r)
    return out[0, 0] / B


def kernel(n_mu, n_sigma2, e_mu, e_sigma2, batch_node_key, batch_node_value,
           batch_edge_key, batch_edge_value):
    vn_r = batch_node_value[:, 0].reshape(R, LANES)
    ve_r = batch_edge_value[:, 0].reshape(R, LANES)

    g_nmu, g_ns2, g_emu, g_es2 = _sc_gather(n_mu, n_sigma2, e_mu, e_sigma2,
                                            batch_node_key, batch_edge_key)
    return _tc_loss(g_nmu.reshape(R, LANES), g_ns2.reshape(R, LANES), vn_r,
                    g_emu.reshape(R, LANES), g_es2.reshape(R, LANES), ve_r)
